# baseline (speedup 1.0000x reference)
import jax
import jax.numpy as jnp
from jax.experimental import pallas as pl
from jax.experimental.pallas import tpu as pltpu

N = 4096
D_IN = 128
HEADS = 4
HID = 16
NC_OUT = 16
NG = 64
EPS = 0.0001

BI = 256
N2 = N // 2
NB2 = N2 // BI


def _half(mask_blk, row0, ib, h_bf, sab_sc, sabt_sc,
          w1_ref, b1_ref, w2_ref, b2_ref):
    bf = jnp.bfloat16
    e2 = jnp.where(mask_blk == 1, 0.5, 0.0).astype(bf)
    sa_bf = sab_sc[pl.ds(row0 + ib * BI, BI), :HEADS]
    sbt_bf = sabt_sc[HEADS:, :]
    base = jnp.dot(e2, h_bf, preferred_element_type=jnp.float32)
    aggs = []
    for hd in range(HEADS):
        t = sa_bf[:, hd:hd + 1] + sbt_bf[hd:hd + 1, :]
        m = jnp.maximum(t, bf(0.01) * t)
        v = jnp.tanh(m)
        wmv = e2 * v
        hh = h_bf[:, hd * HID:(hd + 1) * HID]
        aggs.append(jnp.dot(wmv, hh, preferred_element_type=jnp.float32))
    agg = base + jnp.concatenate(aggs, axis=1)

    z = jax.lax.dot(agg, w1_ref[...]) + b1_ref[...]
    z = jnp.maximum(z, 0.01 * z)
    z = jax.lax.dot(z, w2_ref[...]) + b2_ref[...]
    tmp = jnp.exp(z - jnp.max(z, axis=-1, keepdims=True)) + EPS
    return tmp / jnp.sum(tmp, axis=-1, keepdims=True)


def _main_body(x_ref, wf_ref, phi_ref, mtop_ref, mbot_ref, batw_ref,
               w1_ref, b1_ref, w2_ref, b2_ref,
               logyp_ref, preds_ref,
               h_sc, sab_sc, sabt_sc):
    ib = pl.program_id(0)
    bf = jnp.bfloat16

    @pl.when(ib == 0)
    def _init():
        phi2 = phi_ref[...].reshape(HEADS, 2 * HID)
        r16 = jax.lax.broadcasted_iota(jnp.int32, (HEADS * HID, HEADS), 0) // HID
        gcol = jax.lax.broadcasted_iota(jnp.int32, (HEADS * HID, HEADS), 1)
        u = (r16 == gcol).astype(jnp.float32)
        tmat = jax.lax.dot(u, phi2)
        cidx = jax.lax.broadcasted_iota(jnp.int32, (HEADS * HID, 2 * HID), 1)
        rmod = jax.lax.broadcasted_iota(jnp.int32, (HEADS * HID, 2 * HID), 0) % HID
        va = jnp.sum(jnp.where(cidx == rmod, tmat, 0.0), axis=1, keepdims=True)
        vb = jnp.sum(jnp.where(cidx == rmod + HID, tmat, 0.0), axis=1,
                     keepdims=True)
        pab = 0.5 * jnp.concatenate([va * u, vb * u], axis=1)

        hp = jax.lax.dot(x_ref[...], wf_ref[...])
        sab = jax.lax.dot(hp, pab)
        h_sc[...] = hp.astype(bf)
        sab_sc[...] = sab.astype(bf)
        sabt_sc[...] = jnp.transpose(sab).astype(bf)

    h_bf = h_sc[...]
    preds_ref[pl.ds(ib * BI, BI), :] = _half(
        mtop_ref[0], 0, ib, h_bf, sab_sc, sabt_sc,
        w1_ref, b1_ref, w2_ref, b2_ref)
    preds_ref[pl.ds(N2 + ib * BI, BI), :] = _half(
        mbot_ref[0], N2, ib, h_bf, sab_sc, sabt_sc,
        w1_ref, b1_ref, w2_ref, b2_ref)

    @pl.when(ib == NB2 - 1)
    def _fin():
        bat = batw_ref[...]
        gi = jax.lax.broadcasted_iota(jnp.int32, (NG, N), 0)
        oh = (gi == bat).astype(jnp.float32)
        sums = jax.lax.dot(oh, preds_ref[...])
        cnt = jnp.sum(oh, axis=1, keepdims=True)
        yp = sums / jnp.maximum(cnt, 1.0)
        logyp_ref[...] = jnp.log(yp)


@jax.jit
def kernel(x, batch, mask, Wf, W1, b1, W2, b2, phi):
    mask3 = mask.reshape(2, N2, N)
    batw = batch.reshape(1, N)
    b1r = b1.reshape(1, HID)
    b2r = b2.reshape(1, NC_OUT)

    grid = (NB2,)
    logyp, preds = pl.pallas_call(
        _main_body,
        grid=grid,
        in_specs=[
            pl.BlockSpec((N, D_IN), lambda i: (0, 0)),
            pl.BlockSpec((D_IN, HEADS * HID), lambda i: (0, 0)),
            pl.BlockSpec((HEADS, 2 * HID, 1), lambda i: (0, 0, 0)),
            pl.BlockSpec((1, BI, N), lambda i: (0, i, 0)),
            pl.BlockSpec((1, BI, N), lambda i: (1, i, 0)),
            pl.BlockSpec((1, N), lambda i: (0, 0)),
            pl.BlockSpec((HEADS * HID, HID), lambda i: (0, 0)),
            pl.BlockSpec((1, HID), lambda i: (0, 0)),
            pl.BlockSpec((HID, NC_OUT), lambda i: (0, 0)),
            pl.BlockSpec((1, NC_OUT), lambda i: (0, 0)),
        ],
        out_specs=[
            pl.BlockSpec((NG, NC_OUT), lambda i: (0, 0)),
            pl.BlockSpec((N, NC_OUT), lambda i: (0, 0)),
        ],
        out_shape=[
            jax.ShapeDtypeStruct((NG, NC_OUT), jnp.float32),
            jax.ShapeDtypeStruct((N, NC_OUT), jnp.float32),
        ],
        scratch_shapes=[
            pltpu.VMEM((N, HEADS * HID), jnp.bfloat16),
            pltpu.VMEM((N, 2 * HEADS), jnp.bfloat16),
            pltpu.VMEM((2 * HEADS, N), jnp.bfloat16),
        ],
        compiler_params=pltpu.CompilerParams(
            dimension_semantics=("arbitrary",),
        ),
    )(x, Wf, phi, mask3, mask3, batw, W1, b1r, W2, b2r)

    return (logyp, preds)


# speedup vs baseline: 1.0205x; 1.0205x over previous
import jax
import jax.numpy as jnp
from jax.experimental import pallas as pl
from jax.experimental.pallas import tpu as pltpu

N = 4096
D_IN = 128
HEADS = 4
HID = 16
NC_OUT = 16
NG = 64
EPS = 0.0001

BI = 256
N2 = N // 2
NB2 = N2 // BI


def _half(mask_blk, row0, ib, h_bf, sab_sc, sabt_sc,
          w1_ref, b1_ref, w2_ref, b2_ref):
    bf = jnp.bfloat16
    e2 = jnp.where(mask_blk == 1, 0.5, 0.0).astype(bf)
    sa_bf = sab_sc[pl.ds(row0 + ib * BI, BI), :HEADS]
    sbt_bf = sabt_sc[HEADS:, :]
    base = jnp.dot(e2, h_bf, preferred_element_type=jnp.float32)
    aggs = []
    for hd in range(HEADS):
        t = sa_bf[:, hd:hd + 1] + sbt_bf[hd:hd + 1, :]
        m = jnp.maximum(t, bf(0.01) * t)
        v = jnp.tanh(m)
        wmv = e2 * v
        hh = h_bf[:, hd * HID:(hd + 1) * HID]
        aggs.append(jnp.dot(wmv, hh, preferred_element_type=jnp.float32))
    agg = base + jnp.concatenate(aggs, axis=1)

    z = jax.lax.dot(agg, w1_ref[...]) + b1_ref[...]
    z = jnp.maximum(z, 0.01 * z)
    z = jax.lax.dot(z, w2_ref[...]) + b2_ref[...]
    tmp = jnp.exp(z - jnp.max(z, axis=-1, keepdims=True)) + EPS
    return tmp / jnp.sum(tmp, axis=-1, keepdims=True)


def _main_body(x_ref, wf_ref, phi_ref, mtop_ref, mbot_ref, batw_ref,
               w1_ref, b1_ref, w2_ref, b2_ref,
               logyp_ref, preds_ref,
               h_sc, sab_sc, sabt_sc):
    ib = pl.program_id(0)
    bf = jnp.bfloat16

    @pl.when(ib == 0)
    def _init():
        phi2 = phi_ref[...].reshape(HEADS, 2 * HID)
        r16 = jax.lax.broadcasted_iota(jnp.int32, (HEADS * HID, HEADS), 0) // HID
        gcol = jax.lax.broadcasted_iota(jnp.int32, (HEADS * HID, HEADS), 1)
        u = (r16 == gcol).astype(jnp.float32)
        tmat = jax.lax.dot(u, phi2)
        cidx = jax.lax.broadcasted_iota(jnp.int32, (HEADS * HID, 2 * HID), 1)
        rmod = jax.lax.broadcasted_iota(jnp.int32, (HEADS * HID, 2 * HID), 0) % HID
        va = jnp.sum(jnp.where(cidx == rmod, tmat, 0.0), axis=1, keepdims=True)
        vb = jnp.sum(jnp.where(cidx == rmod + HID, tmat, 0.0), axis=1,
                     keepdims=True)
        pab = 0.5 * jnp.concatenate([va * u, vb * u], axis=1)

        hp = jax.lax.dot(x_ref[...], wf_ref[...])
        sab = jax.lax.dot(hp, pab)
        h_sc[...] = hp.astype(bf)
        sab_sc[...] = sab.astype(bf)
        sabt_sc[...] = jnp.transpose(sab).astype(bf)

    @pl.when(ib > 0)
    def _work():
        jb = ib - 1
        h_bf = h_sc[...]
        preds_ref[pl.ds(jb * BI, BI), :] = _half(
            mtop_ref[0], 0, jb, h_bf, sab_sc, sabt_sc,
            w1_ref, b1_ref, w2_ref, b2_ref)
        preds_ref[pl.ds(N2 + jb * BI, BI), :] = _half(
            mbot_ref[0], N2, jb, h_bf, sab_sc, sabt_sc,
            w1_ref, b1_ref, w2_ref, b2_ref)

    @pl.when(ib == NB2)
    def _fin():
        bat = batw_ref[...]
        gi = jax.lax.broadcasted_iota(jnp.int32, (NG, N), 0)
        oh = (gi == bat).astype(jnp.float32)
        sums = jax.lax.dot(oh, preds_ref[...])
        cnt = jnp.sum(oh, axis=1, keepdims=True)
        yp = sums / jnp.maximum(cnt, 1.0)
        logyp_ref[...] = jnp.log(yp)


@jax.jit
def kernel(x, batch, mask, Wf, W1, b1, W2, b2, phi):
    mask3 = mask.reshape(2, N2, N)
    batw = batch.reshape(1, N)
    b1r = b1.reshape(1, HID)
    b2r = b2.reshape(1, NC_OUT)

    grid = (NB2 + 1,)
    logyp, preds = pl.pallas_call(
        _main_body,
        grid=grid,
        in_specs=[
            pl.BlockSpec((N, D_IN), lambda i: (0, 0)),
            pl.BlockSpec((D_IN, HEADS * HID), lambda i: (0, 0)),
            pl.BlockSpec((HEADS, 2 * HID, 1), lambda i: (0, 0, 0)),
            pl.BlockSpec((1, BI, N),
                         lambda i: (0, jnp.maximum(i - 1, 0), 0)),
            pl.BlockSpec((1, BI, N),
                         lambda i: (1, jnp.maximum(i - 1, 0), 0)),
            pl.BlockSpec((1, N), lambda i: (0, 0)),
            pl.BlockSpec((HEADS * HID, HID), lambda i: (0, 0)),
            pl.BlockSpec((1, HID), lambda i: (0, 0)),
            pl.BlockSpec((HID, NC_OUT), lambda i: (0, 0)),
            pl.BlockSpec((1, NC_OUT), lambda i: (0, 0)),
        ],
        out_specs=[
            pl.BlockSpec((NG, NC_OUT), lambda i: (0, 0)),
            pl.BlockSpec((N, NC_OUT), lambda i: (0, 0)),
        ],
        out_shape=[
            jax.ShapeDtypeStruct((NG, NC_OUT), jnp.float32),
            jax.ShapeDtypeStruct((N, NC_OUT), jnp.float32),
        ],
        scratch_shapes=[
            pltpu.VMEM((N, HEADS * HID), jnp.bfloat16),
            pltpu.VMEM((N, 2 * HEADS), jnp.bfloat16),
            pltpu.VMEM((2 * HEADS, N), jnp.bfloat16),
        ],
        compiler_params=pltpu.CompilerParams(
            dimension_semantics=("arbitrary",),
        ),
    )(x, Wf, phi, mask3, mask3, batw, W1, b1r, W2, b2r)

    return (logyp, preds)
